# baseline (speedup 1.0000x reference)
import functools

import jax
import jax.numpy as jnp
import numpy as np
from jax import lax
from jax.experimental import pallas as pl
from jax.experimental.pallas import tpu as pltpu
from jax.experimental.pallas import tpu_sc as plsc

_NUM_FIELDS = 26
_FIELD_SIZE = 100000
_BATCH = 16384
_NC = 2
_NS = 16
_NW = _NC * _NS
_BPW = _BATCH // _NW
_CHUNK = 128
_NBLK = _BPW // _CHUNK
_NCHUNK = _NBLK * _NUM_FIELDS
_NSL = _CHUNK // 16


def _make_gather_sum():
    mesh = plsc.VectorSubcoreMesh(core_axis_name="c", subcore_axis_name="s")

    @functools.partial(
        pl.kernel,
        mesh=mesh,
        out_type=jax.ShapeDtypeStruct((_BATCH,), jnp.float32),
        scratch_types=[
            pltpu.VMEM((_NCHUNK, _CHUNK), jnp.int32),
            pltpu.VMEM((_CHUNK,), jnp.float32),
            pltpu.VMEM((_BPW,), jnp.float32),
            pltpu.SemaphoreType.DMA,
        ],
    )
    def gather_sum(idx_hbm, table_hbm, out_hbm, idx_v, buf_v, out_v, sem):
        wid = lax.axis_index("s") * _NC + lax.axis_index("c")
        pltpu.sync_copy(idx_hbm.at[wid], idx_v)

        for c in range(_NBLK):
            def body(f, acc, c=c):
                j = c * _NUM_FIELDS + f
                pltpu.async_copy(table_hbm.at[idx_v.at[j]], buf_v, sem).wait()
                return tuple(
                    acc[s] + buf_v[pl.ds(s * 16, 16)] for s in range(_NSL)
                )

            zeros = tuple(
                jnp.zeros((16,), jnp.float32) for _ in range(_NSL)
            )
            acc = lax.fori_loop(0, _NUM_FIELDS, body, zeros)
            for s in range(_NSL):
                out_v[pl.ds(c * _CHUNK + s * 16, 16)] = acc[s]

        pltpu.sync_copy(out_v, out_hbm.at[pl.ds(wid * _BPW, _BPW)])

    return gather_sum


_gather_sum = _make_gather_sum()

_OFFSETS = np.arange(_NUM_FIELDS, dtype=np.int32) * _FIELD_SIZE


def kernel(x, fc_weight, bias):
    idx = x.astype(jnp.int32) + jnp.asarray(_OFFSETS)[None, :]
    arr = (
        idx.reshape(_NW, _NBLK, _CHUNK, _NUM_FIELDS)
        .transpose(0, 1, 3, 2)
        .reshape(_NW, _NCHUNK, _CHUNK)
    )
    table = fc_weight.reshape(-1)
    out = _gather_sum(arr, table)
    return out[:, None] + bias[None, :]


# speedup vs baseline: 1.4100x; 1.4100x over previous
import functools

import jax
import jax.numpy as jnp
import numpy as np
from jax import lax
from jax.experimental import pallas as pl
from jax.experimental.pallas import tpu as pltpu
from jax.experimental.pallas import tpu_sc as plsc

_NUM_FIELDS = 26
_FIELD_SIZE = 100000
_BATCH = 16384
_NC = 2
_NS = 16
_NW = _NC * _NS
_BPW = _BATCH // _NW
_CHUNK = 128
_NBLK = _BPW // _CHUNK
_NCHUNK = _NBLK * _NUM_FIELDS
_NSL = _CHUNK // 16


def _make_gather_sum():
    mesh = plsc.VectorSubcoreMesh(core_axis_name="c", subcore_axis_name="s")

    @functools.partial(
        pl.kernel,
        mesh=mesh,
        out_type=jax.ShapeDtypeStruct((_BATCH,), jnp.float32),
        scratch_types=[
            pltpu.VMEM((_NCHUNK, _CHUNK), jnp.int32),
            pltpu.VMEM((_NCHUNK, _CHUNK), jnp.float32),
            pltpu.VMEM((_BPW,), jnp.float32),
            pltpu.SemaphoreType.DMA,
        ],
    )
    def gather_sum(idx_hbm, table_hbm, out_hbm, idx_v, val_v, out_v, sem):
        wid = lax.axis_index("s") * _NC + lax.axis_index("c")
        pltpu.sync_copy(idx_hbm.at[wid], idx_v)

        def fire(j, _):
            pltpu.async_copy(table_hbm.at[idx_v.at[j]], val_v.at[j], sem)
            return 0

        lax.fori_loop(0, _NCHUNK, fire, 0)

        def drain(j, _):
            pltpu.make_async_copy(
                table_hbm.at[idx_v.at[j]], val_v.at[j], sem
            ).wait()
            return 0

        lax.fori_loop(0, _NCHUNK, drain, 0)

        for c in range(_NBLK):
            def body(f, acc, c=c):
                j = c * _NUM_FIELDS + f
                row = val_v.at[j]
                return tuple(
                    acc[s] + row[pl.ds(s * 16, 16)] for s in range(_NSL)
                )

            zeros = tuple(
                jnp.zeros((16,), jnp.float32) for _ in range(_NSL)
            )
            acc = lax.fori_loop(0, _NUM_FIELDS, body, zeros)
            for s in range(_NSL):
                out_v[pl.ds(c * _CHUNK + s * 16, 16)] = acc[s]

        pltpu.sync_copy(out_v, out_hbm.at[pl.ds(wid * _BPW, _BPW)])

    return gather_sum


_gather_sum = _make_gather_sum()

_OFFSETS = np.arange(_NUM_FIELDS, dtype=np.int32) * _FIELD_SIZE


def kernel(x, fc_weight, bias):
    idx = x.astype(jnp.int32) + jnp.asarray(_OFFSETS)[None, :]
    arr = (
        idx.reshape(_NW, _NBLK, _CHUNK, _NUM_FIELDS)
        .transpose(0, 1, 3, 2)
        .reshape(_NW, _NCHUNK, _CHUNK)
    )
    table = fc_weight.reshape(-1)
    out = _gather_sum(arr, table)
    return out[:, None] + bias[None, :]
